# baseline (speedup 1.0000x reference)
import functools

import jax
import jax.numpy as jnp
from jax import lax
from jax.experimental import pallas as pl
from jax.experimental.pallas import tpu as pltpu
from jax.experimental.pallas import tpu_sc as plsc

VOCAB = 26
NSEG = 15
DM = 512
B = 4096
S = 64
COMBO = 400
NTOK = B * S
PB = 16

NC, NS = 2, 16
NW = NC * NS
TPW = NTOK // NW
G = 64
NG = TPW // G
NBUF = 2
LOOKAHEAD = 1


def _table_body(tok_ref, seg_ref, pos_ref, g_ref, b_ref, out_ref, tokseg_ref):
    @pl.when(pl.program_id(0) == 0)
    def _():
        r_v = lax.broadcasted_iota(jnp.int32, (COMBO, VOCAB), 0)
        c_v = lax.broadcasted_iota(jnp.int32, (COMBO, VOCAB), 1)
        ohv = (r_v // NSEG == c_v).astype(jnp.float32)
        r_g = lax.broadcasted_iota(jnp.int32, (COMBO, NSEG), 0)
        c_g = lax.broadcasted_iota(jnp.int32, (COMBO, NSEG), 1)
        ohg = (r_g % NSEG == c_g).astype(jnp.float32)
        tokseg_ref[...] = lax.dot(
            ohv, tok_ref[...], precision=lax.Precision.HIGHEST
        ) + lax.dot(ohg, seg_ref[...], precision=lax.Precision.HIGHEST)

    pos_blk = pos_ref[pl.ds(pl.program_id(0) * PB, PB), :]
    emb = tokseg_ref[...][None, :, :] + pos_blk[:, None, :]
    mean = jnp.mean(emb, axis=2, keepdims=True)
    var = jnp.mean(emb * emb, axis=2, keepdims=True) - mean * mean
    rstd = lax.rsqrt(var + 1e-5)
    scale = rstd * g_ref[...][None, :, :]
    shift = b_ref[...][None, :, :] - mean * scale
    out_ref[...] = emb * scale + shift


_table_call = pl.pallas_call(
    _table_body,
    grid=(S // PB,),
    in_specs=[
        pl.BlockSpec((VOCAB, DM), lambda s: (0, 0)),
        pl.BlockSpec((NSEG, DM), lambda s: (0, 0)),
        pl.BlockSpec((70, DM), lambda s: (0, 0)),
        pl.BlockSpec((1, DM), lambda s: (0, 0)),
        pl.BlockSpec((1, DM), lambda s: (0, 0)),
    ],
    out_specs=pl.BlockSpec((PB, COMBO, DM), lambda s: (s, 0, 0)),
    out_shape=jax.ShapeDtypeStruct((S, COMBO, DM), jnp.float32),
    scratch_shapes=[pltpu.VMEM((COMBO, DM), jnp.float32)],
)


def _cidx_body(x_ref, seg_ref, out_ref):
    pos = lax.broadcasted_iota(jnp.int32, (B, S), 1)
    out_ref[...] = COMBO * pos + NSEG * x_ref[...] + seg_ref[...]


_cidx_call = pl.pallas_call(
    _cidx_body,
    out_shape=jax.ShapeDtypeStruct((B, S), jnp.int32),
)


@functools.cache
def _sc_gather_call():
    mesh = plsc.VectorSubcoreMesh(
        core_axis_name="c", subcore_axis_name="s", num_cores=NC, num_subcores=NS
    )

    @functools.partial(
        pl.kernel,
        out_type=jax.ShapeDtypeStruct((NTOK, DM), jnp.float32),
        mesh=mesh,
        scratch_types=[
            pltpu.VMEM((NG, G), jnp.int32),
        ]
        + [pltpu.VMEM((G, DM), jnp.float32) for _ in range(NBUF)]
        + [pltpu.SemaphoreType.DMA for _ in range(2 * NBUF)],
    )
    def _sc_gather(table_hbm, cidx_hbm, out_hbm, idx_v, *bufs_and_sems):
        bufs = bufs_and_sems[:NBUF]
        sem_g = bufs_and_sems[NBUF : 2 * NBUF]
        sem_s = bufs_and_sems[2 * NBUF : 3 * NBUF]
        wid = lax.axis_index("s") * NC + lax.axis_index("c")
        pltpu.sync_copy(cidx_hbm.at[wid], idx_v)
        base = wid * TPW

        def gather(j, b):
            pltpu.async_copy(table_hbm.at[idx_v.at[j]], bufs[b], sem_g[b])

        def drain_g(b):
            pltpu.make_async_copy(table_hbm.at[pl.ds(0, G)], bufs[b], sem_g[b]).wait()

        def store(j, b):
            off = pl.multiple_of(base + j * G, G)
            pltpu.async_copy(bufs[b], out_hbm.at[pl.ds(off, G)], sem_s[b])

        def drain_s(b):
            pltpu.make_async_copy(bufs[b], out_hbm.at[pl.ds(base, G)], sem_s[b]).wait()

        for j in range(LOOKAHEAD):
            gather(j, j % NBUF)

        def block(i, carry):
            j0 = i * NBUF
            for b in range(NBUF):
                j = j0 + b
                jn = j + LOOKAHEAD
                bn = (b + LOOKAHEAD) % NBUF

                @pl.when(jnp.logical_and(jn >= NBUF, jn < NG))
                def _():
                    drain_s(bn)

                @pl.when(jn < NG)
                def _():
                    gather(jn, bn)

                drain_g(b)
                store(j, b)
            return carry

        lax.fori_loop(0, NG // NBUF, block, 0)
        for b in range(NBUF):
            drain_s(b)

    return _sc_gather


def kernel(x, seg, tok_w, pos_w, seg_w, gamma, beta):
    table = _table_call(
        tok_w, seg_w, pos_w, gamma.reshape(1, DM), beta.reshape(1, DM)
    ).reshape(S * COMBO, DM)
    cidx = _cidx_call(x.astype(jnp.int32), seg.astype(jnp.int32))
    out = _sc_gather_call()(table, cidx.reshape(NW, NG, G))
    return out.reshape(B, S, DM)


# speedup vs baseline: 1.0149x; 1.0149x over previous
import functools

import jax
import jax.numpy as jnp
from jax import lax
from jax.experimental import pallas as pl
from jax.experimental.pallas import tpu as pltpu
from jax.experimental.pallas import tpu_sc as plsc

VOCAB = 26
NSEG = 15
DM = 512
B = 4096
S = 64
COMBO = 400
NTOK = B * S
PB = 8

NC, NS = 2, 16
NW = NC * NS
TPW = NTOK // NW
G = 64
NG = TPW // G
NBUF = 2
LOOKAHEAD = 1


def _table_body(tok_ref, seg_ref, pos_ref, g_ref, b_ref, out_ref, tokseg_ref):
    @pl.when(pl.program_id(0) == 0)
    def _():
        r_v = lax.broadcasted_iota(jnp.int32, (COMBO, VOCAB), 0)
        c_v = lax.broadcasted_iota(jnp.int32, (COMBO, VOCAB), 1)
        ohv = (r_v // NSEG == c_v).astype(jnp.float32)
        r_g = lax.broadcasted_iota(jnp.int32, (COMBO, NSEG), 0)
        c_g = lax.broadcasted_iota(jnp.int32, (COMBO, NSEG), 1)
        ohg = (r_g % NSEG == c_g).astype(jnp.float32)
        tokseg_ref[...] = lax.dot(
            ohv, tok_ref[...], precision=lax.Precision.HIGHEST
        ) + lax.dot(ohg, seg_ref[...], precision=lax.Precision.HIGHEST)

    pos_blk = pos_ref[pl.ds(pl.program_id(0) * PB, PB), :]
    emb = tokseg_ref[...][None, :, :] + pos_blk[:, None, :]
    mean = jnp.mean(emb, axis=2, keepdims=True)
    var = jnp.mean(emb * emb, axis=2, keepdims=True) - mean * mean
    rstd = lax.rsqrt(var + 1e-5)
    scale = rstd * g_ref[...][None, :, :]
    shift = b_ref[...][None, :, :] - mean * scale
    out_ref[...] = emb * scale + shift


_table_call = pl.pallas_call(
    _table_body,
    grid=(S // PB,),
    in_specs=[
        pl.BlockSpec((VOCAB, DM), lambda s: (0, 0)),
        pl.BlockSpec((NSEG, DM), lambda s: (0, 0)),
        pl.BlockSpec((70, DM), lambda s: (0, 0)),
        pl.BlockSpec((1, DM), lambda s: (0, 0)),
        pl.BlockSpec((1, DM), lambda s: (0, 0)),
    ],
    out_specs=pl.BlockSpec((PB, COMBO, DM), lambda s: (s, 0, 0)),
    out_shape=jax.ShapeDtypeStruct((S, COMBO, DM), jnp.float32),
    scratch_shapes=[pltpu.VMEM((COMBO, DM), jnp.float32)],
)


def _cidx_body(x_ref, seg_ref, out_ref):
    pos = lax.broadcasted_iota(jnp.int32, (B, S), 1)
    out_ref[...] = COMBO * pos + NSEG * x_ref[...] + seg_ref[...]


_cidx_call = pl.pallas_call(
    _cidx_body,
    out_shape=jax.ShapeDtypeStruct((B, S), jnp.int32),
)


@functools.cache
def _sc_gather_call():
    mesh = plsc.VectorSubcoreMesh(
        core_axis_name="c", subcore_axis_name="s", num_cores=NC, num_subcores=NS
    )

    @functools.partial(
        pl.kernel,
        out_type=jax.ShapeDtypeStruct((NTOK, DM), jnp.float32),
        mesh=mesh,
        scratch_types=[
            pltpu.VMEM((NG, G), jnp.int32),
            pltpu.VMEM((NG, G), jnp.int32),
            pltpu.VMEM((NG, G), jnp.int32),
        ]
        + [pltpu.VMEM((G, DM), jnp.float32) for _ in range(NBUF)]
        + [pltpu.SemaphoreType.DMA for _ in range(2 * NBUF)],
    )
    def _sc_gather(table_hbm, x_hbm, seg_hbm, out_hbm, idx_v, x_v, seg_v, *bufs_and_sems):
        bufs = bufs_and_sems[:NBUF]
        sem_g = bufs_and_sems[NBUF : 2 * NBUF]
        sem_s = bufs_and_sems[2 * NBUF : 3 * NBUF]
        wid = lax.axis_index("s") * NC + lax.axis_index("c")
        pltpu.sync_copy(x_hbm.at[wid], x_v)
        pltpu.sync_copy(seg_hbm.at[wid], seg_v)
        base = wid * TPW

        def make_idx(r, carry):
            for q in range(G // 16):
                posq = (
                    lax.broadcasted_iota(jnp.int32, (16,), 0) + (q * 16)
                ) * COMBO
                sl = pl.ds(q * 16, 16)
                idx_v[r, sl] = posq + NSEG * x_v[r, sl] + seg_v[r, sl]
            return carry

        lax.fori_loop(0, NG, make_idx, 0)

        def gather(j, b):
            pltpu.async_copy(table_hbm.at[idx_v.at[j]], bufs[b], sem_g[b])

        def drain_g(b):
            pltpu.make_async_copy(table_hbm.at[pl.ds(0, G)], bufs[b], sem_g[b]).wait()

        def store(j, b):
            off = pl.multiple_of(base + j * G, G)
            pltpu.async_copy(bufs[b], out_hbm.at[pl.ds(off, G)], sem_s[b])

        def drain_s(b):
            pltpu.make_async_copy(bufs[b], out_hbm.at[pl.ds(base, G)], sem_s[b]).wait()

        for j in range(LOOKAHEAD):
            gather(j, j % NBUF)

        def block(i, carry):
            j0 = i * NBUF
            for b in range(NBUF):
                j = j0 + b
                jn = j + LOOKAHEAD
                bn = (b + LOOKAHEAD) % NBUF

                @pl.when(jnp.logical_and(jn >= NBUF, jn < NG))
                def _():
                    drain_s(bn)

                @pl.when(jn < NG)
                def _():
                    gather(jn, bn)

                drain_g(b)
                store(j, b)
            return carry

        lax.fori_loop(0, NG // NBUF, block, 0)
        for b in range(NBUF):
            drain_s(b)

    return _sc_gather


def kernel(x, seg, tok_w, pos_w, seg_w, gamma, beta):
    table = _table_call(
        tok_w, seg_w, pos_w, gamma.reshape(1, DM), beta.reshape(1, DM)
    ).reshape(S * COMBO, DM)
    out = _sc_gather_call()(
        table,
        x.astype(jnp.int32).reshape(NW, NG, G),
        seg.astype(jnp.int32).reshape(NW, NG, G),
    )
    return out.reshape(B, S, DM)
